# Initial kernel scaffold; baseline (speedup 1.0000x reference)
import jax
import jax.numpy as jnp
from jax.experimental import pallas as pl


def kernel(image, conv1_b, conv1_shift, s0b0_conv1_b, s0b0_conv1_shift, s0b0_conv2_b, s0b0_conv2_shift, s0b1_conv1_b, s0b1_conv1_shift, s0b1_conv2_b, s0b1_conv2_shift, s1b0_conv1_b, s1b0_conv1_shift, s1b0_conv2_b, s1b0_conv2_shift, s1b0_down_b, s1b0_down_shift, s1b1_conv1_b, s1b1_conv1_shift, s1b1_conv2_b, s1b1_conv2_shift, s2b0_conv1_b, s2b0_conv1_shift, s2b0_conv2_b, s2b0_conv2_shift, s2b0_down_b, s2b0_down_shift, s2b1_conv1_b, s2b1_conv1_shift, s2b1_conv2_b, s2b1_conv2_shift, s3b0_conv1_b, s3b0_conv1_shift, s3b0_conv2_b, s3b0_conv2_shift, s3b0_down_b, s3b0_down_shift, s3b1_conv1_b, s3b1_conv1_shift, s3b1_conv2_b, s3b1_conv2_shift, head_b, head_shift):
    raise NotImplementedError("write your pallas kernel here")


# speedup vs baseline: 2.9231x; 2.9231x over previous
import functools

import jax
import jax.numpy as jnp
from jax.experimental import pallas as pl
from jax.experimental.pallas import tpu as pltpu

_VMEM_LIMIT = 56 * 1024 * 1024


def _cdiv(a, b):
    return (a + b - 1) // b


def _conv_tap_kernel(*refs, k, s2d, C, Ho, Wo, relu, has_res):
    if has_res:
        x_ref, b_ref, s_ref, r_ref, o_ref = refs
    else:
        x_ref, b_ref, s_ref, o_ref = refs
        r_ref = None
    x = x_ref[...]
    nb = x.shape[0]
    taps = []
    for di in range(k):
        for dj in range(k):
            if s2d:
                pc = 2 * (di % 2) + (dj % 2)
                a_, b_ = di // 2, dj // 2
                taps.append(x[:, a_:a_ + Ho, b_:b_ + Wo,
                              pc * C:(pc + 1) * C])
            else:
                taps.append(x[:, di:di + Ho, dj:dj + Wo, :])
    a = jnp.concatenate(taps, axis=3).reshape(nb * Ho * Wo, k * k * C)
    y = jnp.dot(a, b_ref[...], preferred_element_type=jnp.float32)
    y = y + s_ref[...]
    y = y.reshape(nb, Ho, Wo, y.shape[-1])
    if r_ref is not None:
        y = y + r_ref[...].astype(jnp.float32)
    if relu:
        y = jnp.maximum(y, 0.0)
    o_ref[...] = y.astype(o_ref.dtype)


def _maxpool_tap_kernel(x_ref, o_ref, *, C, Ho, Wo):
    x = x_ref[...]
    m = None
    for di in range(3):
        for dj in range(3):
            pc = 2 * (di % 2) + (dj % 2)
            a_, b_ = di // 2, dj // 2
            t = x[:, a_:a_ + Ho, b_:b_ + Wo, pc * C:(pc + 1) * C]
            m = t if m is None else jnp.maximum(m, t)
    o_ref[...] = m


def _space_to_depth2(x, pad_val):
    N, H, W, C = x.shape
    xp = jnp.pad(x, ((0, 0), (1, 1), (1, 1), (0, 0)),
                 constant_values=pad_val)
    H2, W2 = (H + 2) // 2, (W + 2) // 2
    xs = xp.reshape(N, H2, 2, W2, 2, C).transpose(0, 1, 3, 2, 4, 5)
    return xs.reshape(N, H2, W2, 4 * C)


def _mm_kernel(a_ref, b_ref, s_ref, o_ref, *, relu):
    y = jnp.dot(a_ref[...], b_ref[...], preferred_element_type=jnp.float32)
    y = y + s_ref[...]
    if relu:
        y = jnp.maximum(y, 0.0)
    o_ref[...] = y.astype(o_ref.dtype)


def _head_kernel(x_ref, b_ref, s_ref, o_ref):
    xm = jnp.mean(x_ref[...].astype(jnp.float32), axis=1)
    y = jnp.dot(xm.astype(jnp.bfloat16), b_ref[...],
                preferred_element_type=jnp.float32)
    o_ref[...] = y + s_ref[...]


def _compiler_params(n_par):
    return pltpu.CompilerParams(
        dimension_semantics=("parallel",) * n_par,
        vmem_limit_bytes=_VMEM_LIMIT)


def conv3x3_fused(x, b, shift, *, stride, relu, residual=None, nb):
    N, H, W, C = x.shape
    Ho = (H + 2 - 3) // stride + 1
    Wo = (W + 2 - 3) // stride + 1
    Cout = b.shape[1]
    if stride == 1:
        xp = jnp.pad(x, ((0, 0), (1, 1), (1, 1), (0, 0)))
        xspec = pl.BlockSpec((nb, H + 2, W + 2, C), lambda i: (i, 0, 0, 0))
    else:
        xp = _space_to_depth2(x, 0.0)
        xspec = pl.BlockSpec((nb, Ho + 1, Wo + 1, 4 * C),
                             lambda i: (i, 0, 0, 0))
    ins = [xp, b, shift]
    in_specs = [
        xspec,
        pl.BlockSpec((9 * C, Cout), lambda i: (0, 0)),
        pl.BlockSpec((1, Cout), lambda i: (0, 0)),
    ]
    if residual is not None:
        ins.append(residual)
        in_specs.append(pl.BlockSpec((nb, Ho, Wo, Cout), lambda i: (i, 0, 0, 0)))
    return pl.pallas_call(
        functools.partial(_conv_tap_kernel, k=3, s2d=stride == 2, C=C,
                          Ho=Ho, Wo=Wo,
                          relu=relu, has_res=residual is not None),
        out_shape=jax.ShapeDtypeStruct((N, Ho, Wo, Cout), jnp.bfloat16),
        grid=(N // nb,),
        in_specs=in_specs,
        out_specs=pl.BlockSpec((nb, Ho, Wo, Cout), lambda i: (i, 0, 0, 0)),
        compiler_params=_compiler_params(1),
    )(*ins)


def maxpool_3x3_s2(x, *, nb):
    N, H, W, C = x.shape
    Ho = (H + 2 - 3) // 2 + 1
    Wo = (W + 2 - 3) // 2 + 1
    xs = _space_to_depth2(x, -jnp.inf)
    return pl.pallas_call(
        functools.partial(_maxpool_tap_kernel, C=C, Ho=Ho, Wo=Wo),
        out_shape=jax.ShapeDtypeStruct((N, Ho, Wo, C), x.dtype),
        grid=(N // nb,),
        in_specs=[pl.BlockSpec((nb, Ho + 1, Wo + 1, 4 * C),
                               lambda i: (i, 0, 0, 0))],
        out_specs=pl.BlockSpec((nb, Ho, Wo, C), lambda i: (i, 0, 0, 0)),
        compiler_params=_compiler_params(1),
    )(xs)


def matmul_fused(a, b, shift, *, relu, out_dtype=jnp.bfloat16, tm=2048):
    M, K = a.shape
    N = b.shape[1]
    tm = min(tm, M)
    Mp = _cdiv(M, tm) * tm
    if Mp != M:
        a = jnp.pad(a, ((0, Mp - M), (0, 0)))
    out = pl.pallas_call(
        functools.partial(_mm_kernel, relu=relu),
        out_shape=jax.ShapeDtypeStruct((Mp, N), out_dtype),
        grid=(Mp // tm,),
        in_specs=[
            pl.BlockSpec((tm, K), lambda i: (i, 0)),
            pl.BlockSpec((K, N), lambda i: (0, 0)),
            pl.BlockSpec((1, N), lambda i: (0, 0)),
        ],
        out_specs=pl.BlockSpec((tm, N), lambda i: (i, 0)),
        compiler_params=_compiler_params(1),
    )(a, b, shift)
    return out[:M] if Mp != M else out


def conv1x1_s2(x, b, shift):
    xs = x[:, ::2, ::2, :]
    N, Ho, Wo, C = xs.shape
    y = matmul_fused(xs.reshape(N * Ho * Wo, C), b, shift, relu=False)
    return y.reshape(N, Ho, Wo, b.shape[1])


def stem_conv(x, b, shift):
    N, H, W, C = x.shape
    k, stride, pad = 7, 2, 3
    xp = jnp.pad(x, ((0, 0), (pad, pad), (pad, pad), (0, 0)))
    Ho = (H + 2 * pad - k) // stride + 1
    Wo = (W + 2 * pad - k) // stride + 1
    cols = []
    for i in range(k):
        for j in range(k):
            cols.append(xp[:, i:i + stride * (Ho - 1) + 1:stride,
                           j:j + stride * (Wo - 1) + 1:stride, :])
    a = jnp.concatenate(cols, axis=-1).reshape(N * Ho * Wo, k * k * C)
    y = matmul_fused(a, b, shift, relu=True)
    return y.reshape(N, Ho, Wo, b.shape[1])


def avgpool_head(x, b, shift):
    N, H, W, C = x.shape
    x3 = x.reshape(N, H * W, C)
    out_n = b.shape[1]
    np_ = _cdiv(out_n, 512) * 512
    if np_ != out_n:
        b = jnp.pad(b, ((0, 0), (0, np_ - out_n)))
        shift = jnp.pad(shift, ((0, 0), (0, np_ - out_n)))
    out = pl.pallas_call(
        _head_kernel,
        out_shape=jax.ShapeDtypeStruct((N, np_), jnp.float32),
        grid=(np_ // 512,),
        in_specs=[
            pl.BlockSpec((N, H * W, C), lambda j: (0, 0, 0)),
            pl.BlockSpec((C, 512), lambda j: (0, j)),
            pl.BlockSpec((1, 512), lambda j: (0, j)),
        ],
        out_specs=pl.BlockSpec((N, 512), lambda j: (0, j)),
        compiler_params=_compiler_params(1),
    )(x3, b, shift)
    return out[:, :out_n] if np_ != out_n else out


def _basic_block(x, c1b, c1s, c2b, c2s, down, *, stride, nb_s1, nb_s2):
    out1 = conv3x3_fused(x, c1b, c1s, stride=stride, relu=True, nb=nb_s2)
    if down is not None:
        identity = conv1x1_s2(x, down[0], down[1])
    else:
        identity = x
    return conv3x3_fused(out1, c2b, c2s, stride=1, relu=True,
                         residual=identity, nb=nb_s1)


def kernel(image, conv1_b, conv1_shift, s0b0_conv1_b, s0b0_conv1_shift, s0b0_conv2_b, s0b0_conv2_shift, s0b1_conv1_b, s0b1_conv1_shift, s0b1_conv2_b, s0b1_conv2_shift, s1b0_conv1_b, s1b0_conv1_shift, s1b0_conv2_b, s1b0_conv2_shift, s1b0_down_b, s1b0_down_shift, s1b1_conv1_b, s1b1_conv1_shift, s1b1_conv2_b, s1b1_conv2_shift, s2b0_conv1_b, s2b0_conv1_shift, s2b0_conv2_b, s2b0_conv2_shift, s2b0_down_b, s2b0_down_shift, s2b1_conv1_b, s2b1_conv1_shift, s2b1_conv2_b, s2b1_conv2_shift, s3b0_conv1_b, s3b0_conv1_shift, s3b0_conv2_b, s3b0_conv2_shift, s3b0_down_b, s3b0_down_shift, s3b1_conv1_b, s3b1_conv1_shift, s3b1_conv2_b, s3b1_conv2_shift, head_b, head_shift):
    x = jnp.transpose(image, (0, 2, 3, 1)).astype(jnp.bfloat16)

    x = stem_conv(x, conv1_b, conv1_shift)
    x = maxpool_3x3_s2(x, nb=2)

    res = x
    x = conv3x3_fused(x, s0b0_conv1_b, s0b0_conv1_shift, stride=1, relu=True,
                      nb=2)
    x = conv3x3_fused(x, s0b0_conv2_b, s0b0_conv2_shift, stride=1, relu=True,
                      residual=res, nb=2)
    res = x
    x = conv3x3_fused(x, s0b1_conv1_b, s0b1_conv1_shift, stride=1, relu=True,
                      nb=2)
    x = conv3x3_fused(x, s0b1_conv2_b, s0b1_conv2_shift, stride=1, relu=True,
                      residual=res, nb=2)

    x = _basic_block(x, s1b0_conv1_b, s1b0_conv1_shift,
                     s1b0_conv2_b, s1b0_conv2_shift,
                     (s1b0_down_b, s1b0_down_shift), stride=2,
                     nb_s1=4, nb_s2=4)
    res = x
    x = conv3x3_fused(x, s1b1_conv1_b, s1b1_conv1_shift, stride=1, relu=True,
                      nb=4)
    x = conv3x3_fused(x, s1b1_conv2_b, s1b1_conv2_shift, stride=1, relu=True,
                      residual=res, nb=4)

    x = _basic_block(x, s2b0_conv1_b, s2b0_conv1_shift,
                     s2b0_conv2_b, s2b0_conv2_shift,
                     (s2b0_down_b, s2b0_down_shift), stride=2,
                     nb_s1=8, nb_s2=8)
    res = x
    x = conv3x3_fused(x, s2b1_conv1_b, s2b1_conv1_shift, stride=1, relu=True,
                      nb=8)
    x = conv3x3_fused(x, s2b1_conv2_b, s2b1_conv2_shift, stride=1, relu=True,
                      residual=res, nb=8)

    x = _basic_block(x, s3b0_conv1_b, s3b0_conv1_shift,
                     s3b0_conv2_b, s3b0_conv2_shift,
                     (s3b0_down_b, s3b0_down_shift), stride=2,
                     nb_s1=16, nb_s2=8)
    res = x
    x = conv3x3_fused(x, s3b1_conv1_b, s3b1_conv1_shift, stride=1, relu=True,
                      nb=16)
    x = conv3x3_fused(x, s3b1_conv2_b, s3b1_conv2_shift, stride=1, relu=True,
                      residual=res, nb=16)

    return avgpool_head(x, head_b, head_shift)


# speedup vs baseline: 26.1911x; 8.9600x over previous
import functools

import jax
import jax.numpy as jnp
from jax.experimental import pallas as pl
from jax.experimental.pallas import tpu as pltpu

_VMEM_LIMIT = 56 * 1024 * 1024


def _cdiv(a, b):
    return (a + b - 1) // b


def _conv_tap_kernel(*refs, k, s2d, C, Ho, Wo, relu, has_res):
    if has_res:
        x_ref, b_ref, s_ref, r_ref, o_ref = refs
    else:
        x_ref, b_ref, s_ref, o_ref = refs
        r_ref = None
    x = x_ref[...]
    nb = x.shape[0]
    taps = []
    for di in range(k):
        for dj in range(k):
            if s2d:
                pc = 2 * (di % 2) + (dj % 2)
                a_, b_ = di // 2, dj // 2
                taps.append(x[:, a_:a_ + Ho, b_:b_ + Wo,
                              pc * C:(pc + 1) * C])
            else:
                taps.append(x[:, di:di + Ho, dj:dj + Wo, :])
    a = jnp.concatenate(taps, axis=3).reshape(nb * Ho * Wo, k * k * C)
    y = jnp.dot(a, b_ref[...], preferred_element_type=jnp.float32)
    y = y + s_ref[...]
    y = y.reshape(nb, Ho, Wo, y.shape[-1])
    if r_ref is not None:
        y = y + r_ref[...].astype(jnp.float32)
    if relu:
        y = jnp.maximum(y, 0.0)
    o_ref[...] = y.astype(o_ref.dtype)


def _maxpool_tap_kernel(x_ref, o_ref, *, C, Ho, Wo):
    x = x_ref[...]
    m = None
    for di in range(3):
        for dj in range(3):
            pc = 2 * (di % 2) + (dj % 2)
            a_, b_ = di // 2, dj // 2
            t = x[:, a_:a_ + Ho, b_:b_ + Wo, pc * C:(pc + 1) * C]
            m = t if m is None else jnp.maximum(m, t)
    o_ref[...] = m


def _space_to_depth2(x, pad_val):
    N, H, W, C = x.shape
    xp = jnp.pad(x, ((0, 0), (1, 1), (1, 1), (0, 0)),
                 constant_values=pad_val)
    H2, W2 = (H + 2) // 2, (W + 2) // 2
    xs = xp.reshape(N, H2, 2, W2, 2, C).transpose(0, 1, 3, 2, 4, 5)
    return xs.reshape(N, H2, W2, 4 * C)


def _stem_pool_kernel(q_ref, b00_ref, b01_ref, b10_ref, b11_ref, s_ref,
                      o_ref, *, Ho, Wo):
    q = q_ref[...]
    nb = q.shape[0]
    Hc, Wc = Ho * 2, Wo * 2
    Hq, Wq = Hc // 2, Wc // 2
    shift = s_ref[...]
    ys = {}
    for (r, s), b_ref in (((0, 0), b00_ref), ((0, 1), b01_ref),
                          ((1, 0), b10_ref), ((1, 1), b11_ref)):
        taps = [q[:, a:a + Hq, b:b + Wq, :]
                for a in range(3) for b in range(3)]
        a_mat = jnp.concatenate(taps, axis=3).reshape(nb * Hq * Wq, 9 * 48)
        y = jnp.dot(a_mat, b_ref[...], preferred_element_type=jnp.float32)
        y = jnp.maximum(y + shift, 0.0).reshape(nb, Hq, Wq, y.shape[-1])
        ys[(r, s)] = y

    def sh_i(y):
        z = jnp.zeros_like(y[:, :1])
        return jnp.concatenate([z, y[:, :-1]], axis=1)

    def sh_j(y):
        z = jnp.zeros_like(y[:, :, :1])
        return jnp.concatenate([z, y[:, :, :-1]], axis=2)

    m = ys[(0, 0)]
    m = jnp.maximum(m, ys[(0, 1)])
    m = jnp.maximum(m, sh_j(ys[(0, 1)]))
    m = jnp.maximum(m, ys[(1, 0)])
    m = jnp.maximum(m, sh_i(ys[(1, 0)]))
    y11 = ys[(1, 1)]
    m = jnp.maximum(m, y11)
    m = jnp.maximum(m, sh_i(y11))
    m = jnp.maximum(m, sh_j(y11))
    m = jnp.maximum(m, sh_i(sh_j(y11)))
    o_ref[...] = m.astype(o_ref.dtype)


def _down_kernel(x_ref, b_ref, s_ref, o_ref, *, Ho, Wo, C):
    x = x_ref[...][:, :Ho, :Wo, 3 * C:4 * C]
    nb = x.shape[0]
    y = jnp.dot(x.reshape(nb * Ho * Wo, C), b_ref[...],
                preferred_element_type=jnp.float32)
    y = y + s_ref[...]
    o_ref[...] = y.reshape(nb, Ho, Wo, y.shape[-1]).astype(o_ref.dtype)


def _mm_kernel(a_ref, b_ref, s_ref, o_ref, *, relu):
    y = jnp.dot(a_ref[...], b_ref[...], preferred_element_type=jnp.float32)
    y = y + s_ref[...]
    if relu:
        y = jnp.maximum(y, 0.0)
    o_ref[...] = y.astype(o_ref.dtype)


def _head_kernel(x_ref, b_ref, s_ref, o_ref):
    xm = jnp.mean(x_ref[...].astype(jnp.float32), axis=1)
    y = jnp.dot(xm.astype(jnp.bfloat16), b_ref[...],
                preferred_element_type=jnp.float32)
    o_ref[...] = y + s_ref[...]


def _compiler_params(n_par):
    return pltpu.CompilerParams(
        dimension_semantics=("parallel",) * n_par,
        vmem_limit_bytes=_VMEM_LIMIT)


def conv3x3_fused(x, b, shift, *, stride, relu, residual=None, nb,
                  x_s2d=None):
    N, H, W, C = x.shape
    Ho = (H + 2 - 3) // stride + 1
    Wo = (W + 2 - 3) // stride + 1
    Cout = b.shape[1]
    if stride == 1:
        xp = jnp.pad(x, ((0, 0), (1, 1), (1, 1), (0, 0)))
        xspec = pl.BlockSpec((nb, H + 2, W + 2, C), lambda i: (i, 0, 0, 0))
    else:
        xp = x_s2d if x_s2d is not None else _space_to_depth2(x, 0.0)
        xspec = pl.BlockSpec((nb, Ho + 1, Wo + 1, 4 * C),
                             lambda i: (i, 0, 0, 0))
    ins = [xp, b, shift]
    in_specs = [
        xspec,
        pl.BlockSpec((9 * C, Cout), lambda i: (0, 0)),
        pl.BlockSpec((1, Cout), lambda i: (0, 0)),
    ]
    if residual is not None:
        ins.append(residual)
        in_specs.append(pl.BlockSpec((nb, Ho, Wo, Cout), lambda i: (i, 0, 0, 0)))
    return pl.pallas_call(
        functools.partial(_conv_tap_kernel, k=3, s2d=stride == 2, C=C,
                          Ho=Ho, Wo=Wo,
                          relu=relu, has_res=residual is not None),
        out_shape=jax.ShapeDtypeStruct((N, Ho, Wo, Cout), jnp.bfloat16),
        grid=(N // nb,),
        in_specs=in_specs,
        out_specs=pl.BlockSpec((nb, Ho, Wo, Cout), lambda i: (i, 0, 0, 0)),
        compiler_params=_compiler_params(1),
    )(*ins)


def maxpool_3x3_s2(x, *, nb):
    N, H, W, C = x.shape
    Ho = (H + 2 - 3) // 2 + 1
    Wo = (W + 2 - 3) // 2 + 1
    xs = _space_to_depth2(x, -jnp.inf)
    return pl.pallas_call(
        functools.partial(_maxpool_tap_kernel, C=C, Ho=Ho, Wo=Wo),
        out_shape=jax.ShapeDtypeStruct((N, Ho, Wo, C), x.dtype),
        grid=(N // nb,),
        in_specs=[pl.BlockSpec((nb, Ho + 1, Wo + 1, 4 * C),
                               lambda i: (i, 0, 0, 0))],
        out_specs=pl.BlockSpec((nb, Ho, Wo, C), lambda i: (i, 0, 0, 0)),
        compiler_params=_compiler_params(1),
    )(xs)


def matmul_fused(a, b, shift, *, relu, out_dtype=jnp.bfloat16, tm=2048):
    M, K = a.shape
    N = b.shape[1]
    tm = min(tm, M)
    Mp = _cdiv(M, tm) * tm
    if Mp != M:
        a = jnp.pad(a, ((0, Mp - M), (0, 0)))
    out = pl.pallas_call(
        functools.partial(_mm_kernel, relu=relu),
        out_shape=jax.ShapeDtypeStruct((Mp, N), out_dtype),
        grid=(Mp // tm,),
        in_specs=[
            pl.BlockSpec((tm, K), lambda i: (i, 0)),
            pl.BlockSpec((K, N), lambda i: (0, 0)),
            pl.BlockSpec((1, N), lambda i: (0, 0)),
        ],
        out_specs=pl.BlockSpec((tm, N), lambda i: (i, 0)),
        compiler_params=_compiler_params(1),
    )(a, b, shift)
    return out[:M] if Mp != M else out


def conv1x1_s2(xs2d, b, shift, *, Ho, Wo, C, nb):
    N = xs2d.shape[0]
    Cout = b.shape[1]
    return pl.pallas_call(
        functools.partial(_down_kernel, Ho=Ho, Wo=Wo, C=C),
        out_shape=jax.ShapeDtypeStruct((N, Ho, Wo, Cout), jnp.bfloat16),
        grid=(N // nb,),
        in_specs=[
            pl.BlockSpec((nb, Ho + 1, Wo + 1, 4 * C), lambda i: (i, 0, 0, 0)),
            pl.BlockSpec((C, Cout), lambda i: (0, 0)),
            pl.BlockSpec((1, Cout), lambda i: (0, 0)),
        ],
        out_specs=pl.BlockSpec((nb, Ho, Wo, Cout), lambda i: (i, 0, 0, 0)),
        compiler_params=_compiler_params(1),
    )(xs2d, b, shift)


def _stem_phase_weights(b):
    bp = jnp.concatenate([b, jnp.zeros((1, b.shape[1]), b.dtype)], axis=0)
    mats = []
    for r in (0, 1):
        for s in (0, 1):
            rows = []
            for a in range(3):
                for bb in range(3):
                    for gh in range(4):
                        for gw in range(4):
                            for c in range(3):
                                di = 4 * a + gh - 2 * r
                                dj = 4 * bb + gw - 2 * s
                                if 0 <= di < 7 and 0 <= dj < 7:
                                    rows.append((di * 7 + dj) * 3 + c)
                                else:
                                    rows.append(147)
            mats.append(bp[jnp.array(rows)])
    return mats


def stem_conv_pool(image, b, shift, *, nb):
    N = image.shape[0]
    Ho, Wo = 56, 56
    xp = jnp.pad(image, ((0, 0), (0, 0), (3, 5), (3, 5)))
    q = xp.reshape(N, 3, 58, 4, 58, 4).transpose(0, 2, 4, 3, 5, 1)
    q = q.reshape(N, 58, 58, 48).astype(jnp.bfloat16)
    b00, b01, b10, b11 = _stem_phase_weights(b)
    wspec = pl.BlockSpec((432, 64), lambda i: (0, 0))
    return pl.pallas_call(
        functools.partial(_stem_pool_kernel, Ho=Ho, Wo=Wo),
        out_shape=jax.ShapeDtypeStruct((N, Ho, Wo, 64), jnp.bfloat16),
        grid=(N // nb,),
        in_specs=[
            pl.BlockSpec((nb, 58, 58, 48), lambda i: (i, 0, 0, 0)),
            wspec, wspec, wspec, wspec,
            pl.BlockSpec((1, 64), lambda i: (0, 0)),
        ],
        out_specs=pl.BlockSpec((nb, Ho, Wo, 64), lambda i: (i, 0, 0, 0)),
        compiler_params=_compiler_params(1),
    )(q, b00, b01, b10, b11, shift)


def avgpool_head(x, b, shift):
    N, H, W, C = x.shape
    x3 = x.reshape(N, H * W, C)
    out_n = b.shape[1]
    np_ = _cdiv(out_n, 512) * 512
    if np_ != out_n:
        b = jnp.pad(b, ((0, 0), (0, np_ - out_n)))
        shift = jnp.pad(shift, ((0, 0), (0, np_ - out_n)))
    out = pl.pallas_call(
        _head_kernel,
        out_shape=jax.ShapeDtypeStruct((N, np_), jnp.float32),
        grid=(np_ // 512,),
        in_specs=[
            pl.BlockSpec((N, H * W, C), lambda j: (0, 0, 0)),
            pl.BlockSpec((C, 512), lambda j: (0, j)),
            pl.BlockSpec((1, 512), lambda j: (0, j)),
        ],
        out_specs=pl.BlockSpec((N, 512), lambda j: (0, j)),
        compiler_params=_compiler_params(1),
    )(x3, b, shift)
    return out[:, :out_n] if np_ != out_n else out


def _basic_block(x, c1b, c1s, c2b, c2s, down, *, nb_s1, nb_s2):
    N, H, W, C = x.shape
    xs = _space_to_depth2(x, 0.0)
    out1 = conv3x3_fused(x, c1b, c1s, stride=2, relu=True, nb=nb_s2,
                         x_s2d=xs)
    identity = conv1x1_s2(xs, down[0], down[1],
                          Ho=H // 2, Wo=W // 2, C=C, nb=nb_s2)
    return conv3x3_fused(out1, c2b, c2s, stride=1, relu=True,
                         residual=identity, nb=nb_s1)


def kernel(image, conv1_b, conv1_shift, s0b0_conv1_b, s0b0_conv1_shift, s0b0_conv2_b, s0b0_conv2_shift, s0b1_conv1_b, s0b1_conv1_shift, s0b1_conv2_b, s0b1_conv2_shift, s1b0_conv1_b, s1b0_conv1_shift, s1b0_conv2_b, s1b0_conv2_shift, s1b0_down_b, s1b0_down_shift, s1b1_conv1_b, s1b1_conv1_shift, s1b1_conv2_b, s1b1_conv2_shift, s2b0_conv1_b, s2b0_conv1_shift, s2b0_conv2_b, s2b0_conv2_shift, s2b0_down_b, s2b0_down_shift, s2b1_conv1_b, s2b1_conv1_shift, s2b1_conv2_b, s2b1_conv2_shift, s3b0_conv1_b, s3b0_conv1_shift, s3b0_conv2_b, s3b0_conv2_shift, s3b0_down_b, s3b0_down_shift, s3b1_conv1_b, s3b1_conv1_shift, s3b1_conv2_b, s3b1_conv2_shift, head_b, head_shift):
    x = stem_conv_pool(image, conv1_b, conv1_shift, nb=2)

    res = x
    x = conv3x3_fused(x, s0b0_conv1_b, s0b0_conv1_shift, stride=1, relu=True,
                      nb=2)
    x = conv3x3_fused(x, s0b0_conv2_b, s0b0_conv2_shift, stride=1, relu=True,
                      residual=res, nb=2)
    res = x
    x = conv3x3_fused(x, s0b1_conv1_b, s0b1_conv1_shift, stride=1, relu=True,
                      nb=2)
    x = conv3x3_fused(x, s0b1_conv2_b, s0b1_conv2_shift, stride=1, relu=True,
                      residual=res, nb=2)

    x = _basic_block(x, s1b0_conv1_b, s1b0_conv1_shift,
                     s1b0_conv2_b, s1b0_conv2_shift,
                     (s1b0_down_b, s1b0_down_shift),
                     nb_s1=4, nb_s2=4)
    res = x
    x = conv3x3_fused(x, s1b1_conv1_b, s1b1_conv1_shift, stride=1, relu=True,
                      nb=4)
    x = conv3x3_fused(x, s1b1_conv2_b, s1b1_conv2_shift, stride=1, relu=True,
                      residual=res, nb=4)

    x = _basic_block(x, s2b0_conv1_b, s2b0_conv1_shift,
                     s2b0_conv2_b, s2b0_conv2_shift,
                     (s2b0_down_b, s2b0_down_shift),
                     nb_s1=8, nb_s2=8)
    res = x
    x = conv3x3_fused(x, s2b1_conv1_b, s2b1_conv1_shift, stride=1, relu=True,
                      nb=8)
    x = conv3x3_fused(x, s2b1_conv2_b, s2b1_conv2_shift, stride=1, relu=True,
                      residual=res, nb=8)

    x = _basic_block(x, s3b0_conv1_b, s3b0_conv1_shift,
                     s3b0_conv2_b, s3b0_conv2_shift,
                     (s3b0_down_b, s3b0_down_shift),
                     nb_s1=16, nb_s2=8)
    res = x
    x = conv3x3_fused(x, s3b1_conv1_b, s3b1_conv1_shift, stride=1, relu=True,
                      nb=16)
    x = conv3x3_fused(x, s3b1_conv2_b, s3b1_conv2_shift, stride=1, relu=True,
                      residual=res, nb=16)

    return avgpool_head(x, head_b, head_shift)


# speedup vs baseline: 34.3863x; 1.3129x over previous
import functools

import jax
import jax.numpy as jnp
from jax.experimental import pallas as pl
from jax.experimental.pallas import tpu as pltpu

_VMEM_LIMIT = 56 * 1024 * 1024


def _cdiv(a, b):
    return (a + b - 1) // b


def _pad_ring(y):
    return jnp.pad(y, ((0, 0), (1, 1), (1, 1), (0, 0)))


def _conv_s1_kernel(*refs, Ho, Wo, relu, has_res):
    if has_res:
        x_ref, b_ref, s_ref, r_ref, o_ref = refs
    else:
        x_ref, b_ref, s_ref, o_ref = refs
        r_ref = None
    x = x_ref[...]
    nb, _, _, C = x.shape
    taps = [x[:, di:di + Ho, dj:dj + Wo, :]
            for di in range(3) for dj in range(3)]
    a = jnp.concatenate(taps, axis=3).reshape(nb * Ho * Wo, 9 * C)
    y = jnp.dot(a, b_ref[...], preferred_element_type=jnp.float32)
    y = y + s_ref[...]
    y = y.reshape(nb, Ho, Wo, y.shape[-1])
    if r_ref is not None:
        y = y + r_ref[...][:, 1:-1, 1:-1, :].astype(jnp.float32)
    if relu:
        y = jnp.maximum(y, 0.0)
    o_ref[...] = _pad_ring(y.astype(o_ref.dtype))


def _conv_s2_kernel(x_ref, b_ref, s_ref, o_ref, *, C, Ho, Wo, relu):
    x = x_ref[...]
    nb = x.shape[0]
    taps = []
    for di in range(3):
        p, a = di % 2, di // 2
        for dj in range(3):
            q, bb = dj % 2, dj // 2
            taps.append(x[:, a:a + Ho, p, bb:bb + Wo, q * C:(q + 1) * C])
    a_mat = jnp.concatenate(taps, axis=3).reshape(nb * Ho * Wo, 9 * C)
    y = jnp.dot(a_mat, b_ref[...], preferred_element_type=jnp.float32)
    y = y + s_ref[...]
    if relu:
        y = jnp.maximum(y, 0.0)
    y = y.reshape(nb, Ho, Wo, y.shape[-1])
    o_ref[...] = _pad_ring(y.astype(o_ref.dtype))


def _down_kernel(x_ref, b_ref, s_ref, o_ref, *, C, Ho, Wo):
    x = x_ref[...]
    nb = x.shape[0]
    xph = x[:, :Ho, 1, :Wo, C:2 * C]
    y = jnp.dot(xph.reshape(nb * Ho * Wo, C), b_ref[...],
                preferred_element_type=jnp.float32)
    y = y + s_ref[...]
    y = y.reshape(nb, Ho, Wo, y.shape[-1])
    o_ref[...] = _pad_ring(y.astype(o_ref.dtype))


def _stem_pool_kernel(q_ref, b00_ref, b01_ref, b10_ref, b11_ref, s_ref,
                      o_ref, *, Ho, Wo):
    q = q_ref[...]
    nb = q.shape[0]
    shift = s_ref[...]
    ys = {}
    for (r, s), b_ref in (((0, 0), b00_ref), ((0, 1), b01_ref),
                          ((1, 0), b10_ref), ((1, 1), b11_ref)):
        taps = [q[:, a:a + Ho, b:b + Wo, :]
                for a in range(3) for b in range(3)]
        a_mat = jnp.concatenate(taps, axis=3).reshape(nb * Ho * Wo, 9 * 48)
        y = jnp.dot(a_mat, b_ref[...], preferred_element_type=jnp.float32)
        y = jnp.maximum(y + shift, 0.0).reshape(nb, Ho, Wo, y.shape[-1])
        ys[(r, s)] = y

    def sh_i(y):
        z = jnp.zeros_like(y[:, :1])
        return jnp.concatenate([z, y[:, :-1]], axis=1)

    def sh_j(y):
        z = jnp.zeros_like(y[:, :, :1])
        return jnp.concatenate([z, y[:, :, :-1]], axis=2)

    m = ys[(0, 0)]
    m = jnp.maximum(m, ys[(0, 1)])
    m = jnp.maximum(m, sh_j(ys[(0, 1)]))
    m = jnp.maximum(m, ys[(1, 0)])
    m = jnp.maximum(m, sh_i(ys[(1, 0)]))
    y11 = ys[(1, 1)]
    m = jnp.maximum(m, y11)
    m = jnp.maximum(m, sh_i(y11))
    m = jnp.maximum(m, sh_j(y11))
    m = jnp.maximum(m, sh_i(sh_j(y11)))
    o_ref[...] = _pad_ring(m.astype(o_ref.dtype))


def _head_kernel(x_ref, b_ref, s_ref, o_ref):
    x = x_ref[...][:, 1:-1, 1:-1, :].astype(jnp.float32)
    xm = jnp.mean(x, axis=(1, 2))
    y = jnp.dot(xm.astype(jnp.bfloat16), b_ref[...],
                preferred_element_type=jnp.float32)
    o_ref[...] = y + s_ref[...]


def _compiler_params(n_par):
    return pltpu.CompilerParams(
        dimension_semantics=("parallel",) * n_par,
        vmem_limit_bytes=_VMEM_LIMIT)


def conv3x3_s1(xp, b, shift, *, relu, residual=None, nb):
    N, Hp, Wp, C = xp.shape
    Ho, Wo = Hp - 2, Wp - 2
    Cout = b.shape[1]
    ins = [xp, b, shift]
    in_specs = [
        pl.BlockSpec((nb, Hp, Wp, C), lambda i: (i, 0, 0, 0)),
        pl.BlockSpec((9 * C, Cout), lambda i: (0, 0)),
        pl.BlockSpec((1, Cout), lambda i: (0, 0)),
    ]
    if residual is not None:
        ins.append(residual)
        in_specs.append(pl.BlockSpec((nb, Ho + 2, Wo + 2, Cout),
                                     lambda i: (i, 0, 0, 0)))
    return pl.pallas_call(
        functools.partial(_conv_s1_kernel, Ho=Ho, Wo=Wo, relu=relu,
                          has_res=residual is not None),
        out_shape=jax.ShapeDtypeStruct((N, Ho + 2, Wo + 2, Cout),
                                       jnp.bfloat16),
        grid=(N // nb,),
        in_specs=in_specs,
        out_specs=pl.BlockSpec((nb, Ho + 2, Wo + 2, Cout),
                               lambda i: (i, 0, 0, 0)),
        compiler_params=_compiler_params(1),
    )(*ins)


def _fold_phases(xp):
    N, Hp, Wp, C = xp.shape
    return xp.reshape(N, Hp // 2, 2, Wp // 2, 2 * C)


def conv3x3_s2(xp, b, shift, *, relu, nb):
    N, Hp, Wp, C = xp.shape
    Ho, Wo = (Hp - 2) // 2, (Wp - 2) // 2
    Cout = b.shape[1]
    xf = _fold_phases(xp)
    return pl.pallas_call(
        functools.partial(_conv_s2_kernel, C=C, Ho=Ho, Wo=Wo, relu=relu),
        out_shape=jax.ShapeDtypeStruct((N, Ho + 2, Wo + 2, Cout),
                                       jnp.bfloat16),
        grid=(N // nb,),
        in_specs=[
            pl.BlockSpec((nb, Hp // 2, 2, Wp // 2, 2 * C),
                         lambda i: (i, 0, 0, 0, 0)),
            pl.BlockSpec((9 * C, Cout), lambda i: (0, 0)),
            pl.BlockSpec((1, Cout), lambda i: (0, 0)),
        ],
        out_specs=pl.BlockSpec((nb, Ho + 2, Wo + 2, Cout),
                               lambda i: (i, 0, 0, 0)),
        compiler_params=_compiler_params(1),
    )(xf, b, shift)


def conv1x1_s2(xp, b, shift, *, nb):
    N, Hp, Wp, C = xp.shape
    Ho, Wo = (Hp - 2) // 2, (Wp - 2) // 2
    Cout = b.shape[1]
    xf = _fold_phases(xp)
    return pl.pallas_call(
        functools.partial(_down_kernel, C=C, Ho=Ho, Wo=Wo),
        out_shape=jax.ShapeDtypeStruct((N, Ho + 2, Wo + 2, Cout),
                                       jnp.bfloat16),
        grid=(N // nb,),
        in_specs=[
            pl.BlockSpec((nb, Hp // 2, 2, Wp // 2, 2 * C),
                         lambda i: (i, 0, 0, 0, 0)),
            pl.BlockSpec((C, Cout), lambda i: (0, 0)),
            pl.BlockSpec((1, Cout), lambda i: (0, 0)),
        ],
        out_specs=pl.BlockSpec((nb, Ho + 2, Wo + 2, Cout),
                               lambda i: (i, 0, 0, 0)),
        compiler_params=_compiler_params(1),
    )(xf, b, shift)


def _stem_phase_weights(b):
    bp = jnp.concatenate([b, jnp.zeros((1, b.shape[1]), b.dtype)], axis=0)
    mats = []
    for r in (0, 1):
        for s in (0, 1):
            rows = []
            for a in range(3):
                for bb in range(3):
                    for gh in range(4):
                        for gw in range(4):
                            for c in range(3):
                                di = 4 * a + gh - 2 * r
                                dj = 4 * bb + gw - 2 * s
                                if 0 <= di < 7 and 0 <= dj < 7:
                                    rows.append((di * 7 + dj) * 3 + c)
                                else:
                                    rows.append(147)
            mats.append(bp[jnp.array(rows)])
    return mats


def stem_conv_pool(image, b, shift, *, nb):
    N = image.shape[0]
    Ho, Wo = 56, 56
    xp = jnp.pad(image, ((0, 0), (0, 0), (3, 5), (3, 5)))
    q = xp.reshape(N, 3, 58, 4, 58, 4).transpose(0, 2, 4, 3, 5, 1)
    q = q.reshape(N, 58, 58, 48).astype(jnp.bfloat16)
    b00, b01, b10, b11 = _stem_phase_weights(b)
    wspec = pl.BlockSpec((432, 64), lambda i: (0, 0))
    return pl.pallas_call(
        functools.partial(_stem_pool_kernel, Ho=Ho, Wo=Wo),
        out_shape=jax.ShapeDtypeStruct((N, Ho + 2, Wo + 2, 64), jnp.bfloat16),
        grid=(N // nb,),
        in_specs=[
            pl.BlockSpec((nb, 58, 58, 48), lambda i: (i, 0, 0, 0)),
            wspec, wspec, wspec, wspec,
            pl.BlockSpec((1, 64), lambda i: (0, 0)),
        ],
        out_specs=pl.BlockSpec((nb, Ho + 2, Wo + 2, 64),
                               lambda i: (i, 0, 0, 0)),
        compiler_params=_compiler_params(1),
    )(q, b00, b01, b10, b11, shift)


def avgpool_head(xp, b, shift):
    N, Hp, Wp, C = xp.shape
    out_n = b.shape[1]
    np_ = _cdiv(out_n, 512) * 512
    if np_ != out_n:
        b = jnp.pad(b, ((0, 0), (0, np_ - out_n)))
        shift = jnp.pad(shift, ((0, 0), (0, np_ - out_n)))
    out = pl.pallas_call(
        _head_kernel,
        out_shape=jax.ShapeDtypeStruct((N, np_), jnp.float32),
        grid=(np_ // 512,),
        in_specs=[
            pl.BlockSpec((N, Hp, Wp, C), lambda j: (0, 0, 0, 0)),
            pl.BlockSpec((C, 512), lambda j: (0, j)),
            pl.BlockSpec((1, 512), lambda j: (0, j)),
        ],
        out_specs=pl.BlockSpec((N, 512), lambda j: (0, j)),
        compiler_params=_compiler_params(1),
    )(xp, b, shift)
    return out[:, :out_n] if np_ != out_n else out


def _down_block(x, c1b, c1s, c2b, c2s, down_b, down_s, *, nb_s1, nb_s2):
    out1 = conv3x3_s2(x, c1b, c1s, relu=True, nb=nb_s2)
    identity = conv1x1_s2(x, down_b, down_s, nb=nb_s2)
    return conv3x3_s1(out1, c2b, c2s, relu=True, residual=identity, nb=nb_s1)


def _plain_block(x, c1b, c1s, c2b, c2s, *, nb):
    out1 = conv3x3_s1(x, c1b, c1s, relu=True, nb=nb)
    return conv3x3_s1(out1, c2b, c2s, relu=True, residual=x, nb=nb)


def kernel(image, conv1_b, conv1_shift, s0b0_conv1_b, s0b0_conv1_shift, s0b0_conv2_b, s0b0_conv2_shift, s0b1_conv1_b, s0b1_conv1_shift, s0b1_conv2_b, s0b1_conv2_shift, s1b0_conv1_b, s1b0_conv1_shift, s1b0_conv2_b, s1b0_conv2_shift, s1b0_down_b, s1b0_down_shift, s1b1_conv1_b, s1b1_conv1_shift, s1b1_conv2_b, s1b1_conv2_shift, s2b0_conv1_b, s2b0_conv1_shift, s2b0_conv2_b, s2b0_conv2_shift, s2b0_down_b, s2b0_down_shift, s2b1_conv1_b, s2b1_conv1_shift, s2b1_conv2_b, s2b1_conv2_shift, s3b0_conv1_b, s3b0_conv1_shift, s3b0_conv2_b, s3b0_conv2_shift, s3b0_down_b, s3b0_down_shift, s3b1_conv1_b, s3b1_conv1_shift, s3b1_conv2_b, s3b1_conv2_shift, head_b, head_shift):
    x = stem_conv_pool(image, conv1_b, conv1_shift, nb=2)

    x = _plain_block(x, s0b0_conv1_b, s0b0_conv1_shift,
                     s0b0_conv2_b, s0b0_conv2_shift, nb=2)
    x = _plain_block(x, s0b1_conv1_b, s0b1_conv1_shift,
                     s0b1_conv2_b, s0b1_conv2_shift, nb=2)

    x = _down_block(x, s1b0_conv1_b, s1b0_conv1_shift,
                    s1b0_conv2_b, s1b0_conv2_shift,
                    s1b0_down_b, s1b0_down_shift, nb_s1=4, nb_s2=4)
    x = _plain_block(x, s1b1_conv1_b, s1b1_conv1_shift,
                     s1b1_conv2_b, s1b1_conv2_shift, nb=4)

    x = _down_block(x, s2b0_conv1_b, s2b0_conv1_shift,
                    s2b0_conv2_b, s2b0_conv2_shift,
                    s2b0_down_b, s2b0_down_shift, nb_s1=8, nb_s2=8)
    x = _plain_block(x, s2b1_conv1_b, s2b1_conv1_shift,
                     s2b1_conv2_b, s2b1_conv2_shift, nb=8)

    x = _down_block(x, s3b0_conv1_b, s3b0_conv1_shift,
                    s3b0_conv2_b, s3b0_conv2_shift,
                    s3b0_down_b, s3b0_down_shift, nb_s1=16, nb_s2=8)
    x = _plain_block(x, s3b1_conv1_b, s3b1_conv1_shift,
                     s3b1_conv2_b, s3b1_conv2_shift, nb=16)

    return avgpool_head(x, head_b, head_shift)


# speedup vs baseline: 34.5286x; 1.0041x over previous
import functools

import jax
import jax.numpy as jnp
from jax.experimental import pallas as pl
from jax.experimental.pallas import tpu as pltpu

_VMEM_LIMIT = 56 * 1024 * 1024


def _cdiv(a, b):
    return (a + b - 1) // b


def _pad_ring(y):
    return jnp.pad(y, ((0, 0), (1, 1), (1, 1), (0, 0)))


def _conv_s1_kernel(*refs, Ho, Wo, relu, has_res):
    if has_res:
        x_ref, b_ref, s_ref, r_ref, o_ref = refs
    else:
        x_ref, b_ref, s_ref, o_ref = refs
        r_ref = None
    x = x_ref[...]
    nb, _, _, C = x.shape
    taps = [x[:, di:di + Ho, dj:dj + Wo, :]
            for di in range(3) for dj in range(3)]
    a = jnp.concatenate(taps, axis=3).reshape(nb * Ho * Wo, 9 * C)
    y = jnp.dot(a, b_ref[...], preferred_element_type=jnp.float32)
    y = y + s_ref[...]
    y = y.reshape(nb, Ho, Wo, y.shape[-1])
    if r_ref is not None:
        y = y + r_ref[...][:, 1:-1, 1:-1, :].astype(jnp.float32)
    if relu:
        y = jnp.maximum(y, 0.0)
    o_ref[...] = _pad_ring(y.astype(o_ref.dtype))


def _conv_s2_kernel(x_ref, b_ref, s_ref, o_ref, *, C, Ho, Wo, relu):
    x = x_ref[...]
    nb = x.shape[0]
    taps = []
    for di in range(3):
        p, a = di % 2, di // 2
        for dj in range(3):
            q, bb = dj % 2, dj // 2
            taps.append(x[:, a:a + Ho, p, bb:bb + Wo, q * C:(q + 1) * C])
    a_mat = jnp.concatenate(taps, axis=3).reshape(nb * Ho * Wo, 9 * C)
    y = jnp.dot(a_mat, b_ref[...], preferred_element_type=jnp.float32)
    y = y + s_ref[...]
    if relu:
        y = jnp.maximum(y, 0.0)
    y = y.reshape(nb, Ho, Wo, y.shape[-1])
    o_ref[...] = _pad_ring(y.astype(o_ref.dtype))


def _down_kernel(x_ref, b_ref, s_ref, o_ref, *, C, Ho, Wo):
    x = x_ref[...]
    nb = x.shape[0]
    xph = x[:, :Ho, 1, :Wo, C:2 * C]
    y = jnp.dot(xph.reshape(nb * Ho * Wo, C), b_ref[...],
                preferred_element_type=jnp.float32)
    y = y + s_ref[...]
    y = y.reshape(nb, Ho, Wo, y.shape[-1])
    o_ref[...] = _pad_ring(y.astype(o_ref.dtype))


def _stem_pool_kernel(q_ref, b_ref, s_ref, o_ref, *, Ho, Wo):
    q = q_ref[...]
    nb = q.shape[0]
    shift = s_ref[...]
    taps = [q[:, a:a + Ho, b:b + Wo, :]
            for a in range(3) for b in range(3)]
    a_mat = jnp.concatenate(taps, axis=3).reshape(nb * Ho * Wo, 9 * 48)
    y4 = jnp.dot(a_mat, b_ref[...], preferred_element_type=jnp.float32)
    ys = {}
    for k, (r, s) in enumerate(((0, 0), (0, 1), (1, 0), (1, 1))):
        y = y4[:, k * 64:(k + 1) * 64]
        y = jnp.maximum(y + shift, 0.0).reshape(nb, Ho, Wo, 64)
        ys[(r, s)] = y

    def sh_i(y):
        z = jnp.zeros_like(y[:, :1])
        return jnp.concatenate([z, y[:, :-1]], axis=1)

    def sh_j(y):
        z = jnp.zeros_like(y[:, :, :1])
        return jnp.concatenate([z, y[:, :, :-1]], axis=2)

    m = ys[(0, 0)]
    m = jnp.maximum(m, ys[(0, 1)])
    m = jnp.maximum(m, sh_j(ys[(0, 1)]))
    m = jnp.maximum(m, ys[(1, 0)])
    m = jnp.maximum(m, sh_i(ys[(1, 0)]))
    y11 = ys[(1, 1)]
    m = jnp.maximum(m, y11)
    m = jnp.maximum(m, sh_i(y11))
    m = jnp.maximum(m, sh_j(y11))
    m = jnp.maximum(m, sh_i(sh_j(y11)))
    o_ref[...] = _pad_ring(m.astype(o_ref.dtype))


def _head_kernel(x_ref, b_ref, s_ref, o_ref):
    x = x_ref[...][:, 1:-1, 1:-1, :].astype(jnp.float32)
    xm = jnp.mean(x, axis=(1, 2))
    y = jnp.dot(xm.astype(jnp.bfloat16), b_ref[...],
                preferred_element_type=jnp.float32)
    o_ref[...] = y + s_ref[...]


def _compiler_params(n_par):
    return pltpu.CompilerParams(
        dimension_semantics=("parallel",) * n_par,
        vmem_limit_bytes=_VMEM_LIMIT)


def conv3x3_s1(xp, b, shift, *, relu, residual=None, nb):
    N, Hp, Wp, C = xp.shape
    Ho, Wo = Hp - 2, Wp - 2
    Cout = b.shape[1]
    ins = [xp, b, shift]
    in_specs = [
        pl.BlockSpec((nb, Hp, Wp, C), lambda i: (i, 0, 0, 0)),
        pl.BlockSpec((9 * C, Cout), lambda i: (0, 0)),
        pl.BlockSpec((1, Cout), lambda i: (0, 0)),
    ]
    if residual is not None:
        ins.append(residual)
        in_specs.append(pl.BlockSpec((nb, Ho + 2, Wo + 2, Cout),
                                     lambda i: (i, 0, 0, 0)))
    return pl.pallas_call(
        functools.partial(_conv_s1_kernel, Ho=Ho, Wo=Wo, relu=relu,
                          has_res=residual is not None),
        out_shape=jax.ShapeDtypeStruct((N, Ho + 2, Wo + 2, Cout),
                                       jnp.bfloat16),
        grid=(N // nb,),
        in_specs=in_specs,
        out_specs=pl.BlockSpec((nb, Ho + 2, Wo + 2, Cout),
                               lambda i: (i, 0, 0, 0)),
        compiler_params=_compiler_params(1),
    )(*ins)


def _fold_phases(xp):
    N, Hp, Wp, C = xp.shape
    return xp.reshape(N, Hp // 2, 2, Wp // 2, 2 * C)


def conv3x3_s2(xp, b, shift, *, relu, nb):
    N, Hp, Wp, C = xp.shape
    Ho, Wo = (Hp - 2) // 2, (Wp - 2) // 2
    Cout = b.shape[1]
    xf = _fold_phases(xp)
    return pl.pallas_call(
        functools.partial(_conv_s2_kernel, C=C, Ho=Ho, Wo=Wo, relu=relu),
        out_shape=jax.ShapeDtypeStruct((N, Ho + 2, Wo + 2, Cout),
                                       jnp.bfloat16),
        grid=(N // nb,),
        in_specs=[
            pl.BlockSpec((nb, Hp // 2, 2, Wp // 2, 2 * C),
                         lambda i: (i, 0, 0, 0, 0)),
            pl.BlockSpec((9 * C, Cout), lambda i: (0, 0)),
            pl.BlockSpec((1, Cout), lambda i: (0, 0)),
        ],
        out_specs=pl.BlockSpec((nb, Ho + 2, Wo + 2, Cout),
                               lambda i: (i, 0, 0, 0)),
        compiler_params=_compiler_params(1),
    )(xf, b, shift)


def conv1x1_s2(xp, b, shift, *, nb):
    N, Hp, Wp, C = xp.shape
    Ho, Wo = (Hp - 2) // 2, (Wp - 2) // 2
    Cout = b.shape[1]
    xf = _fold_phases(xp)
    return pl.pallas_call(
        functools.partial(_down_kernel, C=C, Ho=Ho, Wo=Wo),
        out_shape=jax.ShapeDtypeStruct((N, Ho + 2, Wo + 2, Cout),
                                       jnp.bfloat16),
        grid=(N // nb,),
        in_specs=[
            pl.BlockSpec((nb, Hp // 2, 2, Wp // 2, 2 * C),
                         lambda i: (i, 0, 0, 0, 0)),
            pl.BlockSpec((C, Cout), lambda i: (0, 0)),
            pl.BlockSpec((1, Cout), lambda i: (0, 0)),
        ],
        out_specs=pl.BlockSpec((nb, Ho + 2, Wo + 2, Cout),
                               lambda i: (i, 0, 0, 0)),
        compiler_params=_compiler_params(1),
    )(xf, b, shift)


def _stem_phase_weights(b):
    bp = jnp.concatenate([b, jnp.zeros((1, b.shape[1]), b.dtype)], axis=0)
    mats = []
    for r in (0, 1):
        for s in (0, 1):
            rows = []
            for a in range(3):
                for bb in range(3):
                    for gh in range(4):
                        for gw in range(4):
                            for c in range(3):
                                di = 4 * a + gh - 2 * r
                                dj = 4 * bb + gw - 2 * s
                                if 0 <= di < 7 and 0 <= dj < 7:
                                    rows.append((di * 7 + dj) * 3 + c)
                                else:
                                    rows.append(147)
            mats.append(bp[jnp.array(rows)])
    return mats


def stem_conv_pool(image, b, shift, *, nb):
    N = image.shape[0]
    Ho, Wo = 56, 56
    xp = jnp.pad(image, ((0, 0), (0, 0), (3, 5), (3, 5)))
    q = xp.reshape(N, 3, 58, 4, 58, 4).transpose(0, 2, 4, 3, 5, 1)
    q = q.reshape(N, 58, 58, 48).astype(jnp.bfloat16)
    b4 = jnp.concatenate(_stem_phase_weights(b), axis=1)
    return pl.pallas_call(
        functools.partial(_stem_pool_kernel, Ho=Ho, Wo=Wo),
        out_shape=jax.ShapeDtypeStruct((N, Ho + 2, Wo + 2, 64), jnp.bfloat16),
        grid=(N // nb,),
        in_specs=[
            pl.BlockSpec((nb, 58, 58, 48), lambda i: (i, 0, 0, 0)),
            pl.BlockSpec((432, 256), lambda i: (0, 0)),
            pl.BlockSpec((1, 64), lambda i: (0, 0)),
        ],
        out_specs=pl.BlockSpec((nb, Ho + 2, Wo + 2, 64),
                               lambda i: (i, 0, 0, 0)),
        compiler_params=_compiler_params(1),
    )(q, b4, shift)


def avgpool_head(xp, b, shift):
    N, Hp, Wp, C = xp.shape
    out_n = b.shape[1]
    np_ = _cdiv(out_n, 512) * 512
    if np_ != out_n:
        b = jnp.pad(b, ((0, 0), (0, np_ - out_n)))
        shift = jnp.pad(shift, ((0, 0), (0, np_ - out_n)))
    out = pl.pallas_call(
        _head_kernel,
        out_shape=jax.ShapeDtypeStruct((N, np_), jnp.float32),
        grid=(np_ // 512,),
        in_specs=[
            pl.BlockSpec((N, Hp, Wp, C), lambda j: (0, 0, 0, 0)),
            pl.BlockSpec((C, 512), lambda j: (0, j)),
            pl.BlockSpec((1, 512), lambda j: (0, j)),
        ],
        out_specs=pl.BlockSpec((N, 512), lambda j: (0, j)),
        compiler_params=_compiler_params(1),
    )(xp, b, shift)
    return out[:, :out_n] if np_ != out_n else out


def _down_block(x, c1b, c1s, c2b, c2s, down_b, down_s, *, nb_s1, nb_s2):
    out1 = conv3x3_s2(x, c1b, c1s, relu=True, nb=nb_s2)
    identity = conv1x1_s2(x, down_b, down_s, nb=nb_s2)
    return conv3x3_s1(out1, c2b, c2s, relu=True, residual=identity, nb=nb_s1)


def _plain_block(x, c1b, c1s, c2b, c2s, *, nb):
    out1 = conv3x3_s1(x, c1b, c1s, relu=True, nb=nb)
    return conv3x3_s1(out1, c2b, c2s, relu=True, residual=x, nb=nb)


def kernel(image, conv1_b, conv1_shift, s0b0_conv1_b, s0b0_conv1_shift, s0b0_conv2_b, s0b0_conv2_shift, s0b1_conv1_b, s0b1_conv1_shift, s0b1_conv2_b, s0b1_conv2_shift, s1b0_conv1_b, s1b0_conv1_shift, s1b0_conv2_b, s1b0_conv2_shift, s1b0_down_b, s1b0_down_shift, s1b1_conv1_b, s1b1_conv1_shift, s1b1_conv2_b, s1b1_conv2_shift, s2b0_conv1_b, s2b0_conv1_shift, s2b0_conv2_b, s2b0_conv2_shift, s2b0_down_b, s2b0_down_shift, s2b1_conv1_b, s2b1_conv1_shift, s2b1_conv2_b, s2b1_conv2_shift, s3b0_conv1_b, s3b0_conv1_shift, s3b0_conv2_b, s3b0_conv2_shift, s3b0_down_b, s3b0_down_shift, s3b1_conv1_b, s3b1_conv1_shift, s3b1_conv2_b, s3b1_conv2_shift, head_b, head_shift):
    x = stem_conv_pool(image, conv1_b, conv1_shift, nb=2)

    x = _plain_block(x, s0b0_conv1_b, s0b0_conv1_shift,
                     s0b0_conv2_b, s0b0_conv2_shift, nb=2)
    x = _plain_block(x, s0b1_conv1_b, s0b1_conv1_shift,
                     s0b1_conv2_b, s0b1_conv2_shift, nb=2)

    x = _down_block(x, s1b0_conv1_b, s1b0_conv1_shift,
                    s1b0_conv2_b, s1b0_conv2_shift,
                    s1b0_down_b, s1b0_down_shift, nb_s1=4, nb_s2=4)
    x = _plain_block(x, s1b1_conv1_b, s1b1_conv1_shift,
                     s1b1_conv2_b, s1b1_conv2_shift, nb=4)

    x = _down_block(x, s2b0_conv1_b, s2b0_conv1_shift,
                    s2b0_conv2_b, s2b0_conv2_shift,
                    s2b0_down_b, s2b0_down_shift, nb_s1=8, nb_s2=8)
    x = _plain_block(x, s2b1_conv1_b, s2b1_conv1_shift,
                     s2b1_conv2_b, s2b1_conv2_shift, nb=8)

    x = _down_block(x, s3b0_conv1_b, s3b0_conv1_shift,
                    s3b0_conv2_b, s3b0_conv2_shift,
                    s3b0_down_b, s3b0_down_shift, nb_s1=16, nb_s2=8)
    x = _plain_block(x, s3b1_conv1_b, s3b1_conv1_shift,
                     s3b1_conv2_b, s3b1_conv2_shift, nb=16)

    return avgpool_head(x, head_b, head_shift)


# speedup vs baseline: 34.7039x; 1.0051x over previous
import functools

import jax
import jax.numpy as jnp
from jax.experimental import pallas as pl
from jax.experimental.pallas import tpu as pltpu

_VMEM_LIMIT = 56 * 1024 * 1024


def _cdiv(a, b):
    return (a + b - 1) // b


def _pad_ring(y):
    return jnp.pad(y, ((0, 0), (1, 1), (1, 1), (0, 0)))


def _conv_s1_kernel(*refs, Ho, Wo, relu, has_res):
    if has_res:
        x_ref, b_ref, s_ref, r_ref, o_ref = refs
    else:
        x_ref, b_ref, s_ref, o_ref = refs
        r_ref = None
    x = x_ref[...]
    nb, _, _, C = x.shape
    taps = [x[:, di:di + Ho, dj:dj + Wo, :]
            for di in range(3) for dj in range(3)]
    a = jnp.concatenate(taps, axis=3).reshape(nb * Ho * Wo, 9 * C)
    y = jnp.dot(a, b_ref[...], preferred_element_type=jnp.float32)
    y = y + s_ref[...]
    y = y.reshape(nb, Ho, Wo, y.shape[-1])
    if r_ref is not None:
        y = y + r_ref[...][:, 1:-1, 1:-1, :].astype(jnp.float32)
    if relu:
        y = jnp.maximum(y, 0.0)
    o_ref[...] = _pad_ring(y.astype(o_ref.dtype))


def _conv_s2_kernel(x_ref, b_ref, s_ref, o_ref, *, C, Ho, Wo, relu):
    x = x_ref[...]
    nb = x.shape[0]
    taps = []
    for di in range(3):
        p, a = di % 2, di // 2
        for dj in range(3):
            q, bb = dj % 2, dj // 2
            taps.append(x[:, a:a + Ho, p, bb:bb + Wo, q * C:(q + 1) * C])
    a_mat = jnp.concatenate(taps, axis=3).reshape(nb * Ho * Wo, 9 * C)
    y = jnp.dot(a_mat, b_ref[...], preferred_element_type=jnp.float32)
    y = y + s_ref[...]
    if relu:
        y = jnp.maximum(y, 0.0)
    y = y.reshape(nb, Ho, Wo, y.shape[-1])
    o_ref[...] = _pad_ring(y.astype(o_ref.dtype))


def _down_kernel(x_ref, b_ref, s_ref, o_ref, *, C, Ho, Wo):
    x = x_ref[...]
    nb = x.shape[0]
    xph = x[:, :Ho, 1, :Wo, C:2 * C]
    y = jnp.dot(xph.reshape(nb * Ho * Wo, C), b_ref[...],
                preferred_element_type=jnp.float32)
    y = y + s_ref[...]
    y = y.reshape(nb, Ho, Wo, y.shape[-1])
    o_ref[...] = _pad_ring(y.astype(o_ref.dtype))


def _stem_pool_kernel(q_ref, b_ref, s_ref, o_ref, *, Ho, Wo):
    q = q_ref[...]
    nb = q.shape[0]
    shift = s_ref[...]
    taps = [q[:, a:a + Ho, b:b + Wo, :]
            for a in range(3) for b in range(3)]
    a_mat = jnp.concatenate(taps, axis=3).reshape(nb * Ho * Wo, 9 * 48)
    y4 = jnp.dot(a_mat, b_ref[...], preferred_element_type=jnp.float32)
    ys = [y4[:, k * 64:(k + 1) * 64].reshape(nb, Ho, Wo, 64)
          for k in range(4)]
    y00, y01, y10, y11 = ys

    def sh_i(y):
        z = jnp.full_like(y[:, :1], -jnp.inf)
        return jnp.concatenate([z, y[:, :-1]], axis=1)

    def sh_j(y):
        z = jnp.full_like(y[:, :, :1], -jnp.inf)
        return jnp.concatenate([z, y[:, :, :-1]], axis=2)

    c0 = jnp.maximum(jnp.maximum(y00, y01), sh_j(y01))
    c1 = jnp.maximum(jnp.maximum(y10, y11), sh_j(y11))
    m = jnp.maximum(jnp.maximum(c0, c1), sh_i(c1))
    m = jnp.maximum(m + shift, 0.0)
    o_ref[...] = _pad_ring(m.astype(o_ref.dtype))


def _head_kernel(x_ref, b_ref, s_ref, o_ref):
    x = x_ref[...][:, 1:-1, 1:-1, :].astype(jnp.float32)
    xm = jnp.mean(x, axis=(1, 2))
    y = jnp.dot(xm.astype(jnp.bfloat16), b_ref[...],
                preferred_element_type=jnp.float32)
    o_ref[...] = y + s_ref[...]


def _compiler_params(n_par):
    return pltpu.CompilerParams(
        dimension_semantics=("parallel",) * n_par,
        vmem_limit_bytes=_VMEM_LIMIT)


def conv3x3_s1(xp, b, shift, *, relu, residual=None, nb):
    N, Hp, Wp, C = xp.shape
    Ho, Wo = Hp - 2, Wp - 2
    Cout = b.shape[1]
    ins = [xp, b, shift]
    in_specs = [
        pl.BlockSpec((nb, Hp, Wp, C), lambda i: (i, 0, 0, 0)),
        pl.BlockSpec((9 * C, Cout), lambda i: (0, 0)),
        pl.BlockSpec((1, Cout), lambda i: (0, 0)),
    ]
    if residual is not None:
        ins.append(residual)
        in_specs.append(pl.BlockSpec((nb, Ho + 2, Wo + 2, Cout),
                                     lambda i: (i, 0, 0, 0)))
    return pl.pallas_call(
        functools.partial(_conv_s1_kernel, Ho=Ho, Wo=Wo, relu=relu,
                          has_res=residual is not None),
        out_shape=jax.ShapeDtypeStruct((N, Ho + 2, Wo + 2, Cout),
                                       jnp.bfloat16),
        grid=(N // nb,),
        in_specs=in_specs,
        out_specs=pl.BlockSpec((nb, Ho + 2, Wo + 2, Cout),
                               lambda i: (i, 0, 0, 0)),
        compiler_params=_compiler_params(1),
    )(*ins)


def _fold_phases(xp):
    N, Hp, Wp, C = xp.shape
    return xp.reshape(N, Hp // 2, 2, Wp // 2, 2 * C)


def conv3x3_s2(xp, b, shift, *, relu, nb):
    N, Hp, Wp, C = xp.shape
    Ho, Wo = (Hp - 2) // 2, (Wp - 2) // 2
    Cout = b.shape[1]
    xf = _fold_phases(xp)
    return pl.pallas_call(
        functools.partial(_conv_s2_kernel, C=C, Ho=Ho, Wo=Wo, relu=relu),
        out_shape=jax.ShapeDtypeStruct((N, Ho + 2, Wo + 2, Cout),
                                       jnp.bfloat16),
        grid=(N // nb,),
        in_specs=[
            pl.BlockSpec((nb, Hp // 2, 2, Wp // 2, 2 * C),
                         lambda i: (i, 0, 0, 0, 0)),
            pl.BlockSpec((9 * C, Cout), lambda i: (0, 0)),
            pl.BlockSpec((1, Cout), lambda i: (0, 0)),
        ],
        out_specs=pl.BlockSpec((nb, Ho + 2, Wo + 2, Cout),
                               lambda i: (i, 0, 0, 0)),
        compiler_params=_compiler_params(1),
    )(xf, b, shift)


def conv1x1_s2(xp, b, shift, *, nb):
    N, Hp, Wp, C = xp.shape
    Ho, Wo = (Hp - 2) // 2, (Wp - 2) // 2
    Cout = b.shape[1]
    xf = _fold_phases(xp)
    return pl.pallas_call(
        functools.partial(_down_kernel, C=C, Ho=Ho, Wo=Wo),
        out_shape=jax.ShapeDtypeStruct((N, Ho + 2, Wo + 2, Cout),
                                       jnp.bfloat16),
        grid=(N // nb,),
        in_specs=[
            pl.BlockSpec((nb, Hp // 2, 2, Wp // 2, 2 * C),
                         lambda i: (i, 0, 0, 0, 0)),
            pl.BlockSpec((C, Cout), lambda i: (0, 0)),
            pl.BlockSpec((1, Cout), lambda i: (0, 0)),
        ],
        out_specs=pl.BlockSpec((nb, Ho + 2, Wo + 2, Cout),
                               lambda i: (i, 0, 0, 0)),
        compiler_params=_compiler_params(1),
    )(xf, b, shift)


def _stem_phase_weights(b):
    bp = jnp.concatenate([b, jnp.zeros((1, b.shape[1]), b.dtype)], axis=0)
    mats = []
    for r in (0, 1):
        for s in (0, 1):
            rows = []
            for a in range(3):
                for bb in range(3):
                    for gh in range(4):
                        for gw in range(4):
                            for c in range(3):
                                di = 4 * a + gh - 2 * r
                                dj = 4 * bb + gw - 2 * s
                                if 0 <= di < 7 and 0 <= dj < 7:
                                    rows.append((di * 7 + dj) * 3 + c)
                                else:
                                    rows.append(147)
            mats.append(bp[jnp.array(rows)])
    return mats


def stem_conv_pool(image, b, shift, *, nb):
    N = image.shape[0]
    Ho, Wo = 56, 56
    xp = jnp.pad(image, ((0, 0), (0, 0), (3, 5), (3, 5)))
    q = xp.reshape(N, 3, 58, 4, 58, 4).transpose(0, 2, 4, 3, 5, 1)
    q = q.reshape(N, 58, 58, 48).astype(jnp.bfloat16)
    b4 = jnp.concatenate(_stem_phase_weights(b), axis=1)
    return pl.pallas_call(
        functools.partial(_stem_pool_kernel, Ho=Ho, Wo=Wo),
        out_shape=jax.ShapeDtypeStruct((N, Ho + 2, Wo + 2, 64), jnp.bfloat16),
        grid=(N // nb,),
        in_specs=[
            pl.BlockSpec((nb, 58, 58, 48), lambda i: (i, 0, 0, 0)),
            pl.BlockSpec((432, 256), lambda i: (0, 0)),
            pl.BlockSpec((1, 64), lambda i: (0, 0)),
        ],
        out_specs=pl.BlockSpec((nb, Ho + 2, Wo + 2, 64),
                               lambda i: (i, 0, 0, 0)),
        compiler_params=_compiler_params(1),
    )(q, b4, shift)


def avgpool_head(xp, b, shift):
    N, Hp, Wp, C = xp.shape
    out_n = b.shape[1]
    np_ = _cdiv(out_n, 512) * 512
    if np_ != out_n:
        b = jnp.pad(b, ((0, 0), (0, np_ - out_n)))
        shift = jnp.pad(shift, ((0, 0), (0, np_ - out_n)))
    out = pl.pallas_call(
        _head_kernel,
        out_shape=jax.ShapeDtypeStruct((N, np_), jnp.float32),
        grid=(np_ // 512,),
        in_specs=[
            pl.BlockSpec((N, Hp, Wp, C), lambda j: (0, 0, 0, 0)),
            pl.BlockSpec((C, 512), lambda j: (0, j)),
            pl.BlockSpec((1, 512), lambda j: (0, j)),
        ],
        out_specs=pl.BlockSpec((N, 512), lambda j: (0, j)),
        compiler_params=_compiler_params(1),
    )(xp, b, shift)
    return out[:, :out_n] if np_ != out_n else out


def _down_block(x, c1b, c1s, c2b, c2s, down_b, down_s, *, nb_s1, nb_s2):
    out1 = conv3x3_s2(x, c1b, c1s, relu=True, nb=nb_s2)
    identity = conv1x1_s2(x, down_b, down_s, nb=nb_s2)
    return conv3x3_s1(out1, c2b, c2s, relu=True, residual=identity, nb=nb_s1)


def _plain_block(x, c1b, c1s, c2b, c2s, *, nb):
    out1 = conv3x3_s1(x, c1b, c1s, relu=True, nb=nb)
    return conv3x3_s1(out1, c2b, c2s, relu=True, residual=x, nb=nb)


def kernel(image, conv1_b, conv1_shift, s0b0_conv1_b, s0b0_conv1_shift, s0b0_conv2_b, s0b0_conv2_shift, s0b1_conv1_b, s0b1_conv1_shift, s0b1_conv2_b, s0b1_conv2_shift, s1b0_conv1_b, s1b0_conv1_shift, s1b0_conv2_b, s1b0_conv2_shift, s1b0_down_b, s1b0_down_shift, s1b1_conv1_b, s1b1_conv1_shift, s1b1_conv2_b, s1b1_conv2_shift, s2b0_conv1_b, s2b0_conv1_shift, s2b0_conv2_b, s2b0_conv2_shift, s2b0_down_b, s2b0_down_shift, s2b1_conv1_b, s2b1_conv1_shift, s2b1_conv2_b, s2b1_conv2_shift, s3b0_conv1_b, s3b0_conv1_shift, s3b0_conv2_b, s3b0_conv2_shift, s3b0_down_b, s3b0_down_shift, s3b1_conv1_b, s3b1_conv1_shift, s3b1_conv2_b, s3b1_conv2_shift, head_b, head_shift):
    x = stem_conv_pool(image, conv1_b, conv1_shift, nb=2)

    x = _plain_block(x, s0b0_conv1_b, s0b0_conv1_shift,
                     s0b0_conv2_b, s0b0_conv2_shift, nb=2)
    x = _plain_block(x, s0b1_conv1_b, s0b1_conv1_shift,
                     s0b1_conv2_b, s0b1_conv2_shift, nb=2)

    x = _down_block(x, s1b0_conv1_b, s1b0_conv1_shift,
                    s1b0_conv2_b, s1b0_conv2_shift,
                    s1b0_down_b, s1b0_down_shift, nb_s1=4, nb_s2=4)
    x = _plain_block(x, s1b1_conv1_b, s1b1_conv1_shift,
                     s1b1_conv2_b, s1b1_conv2_shift, nb=4)

    x = _down_block(x, s2b0_conv1_b, s2b0_conv1_shift,
                    s2b0_conv2_b, s2b0_conv2_shift,
                    s2b0_down_b, s2b0_down_shift, nb_s1=8, nb_s2=8)
    x = _plain_block(x, s2b1_conv1_b, s2b1_conv1_shift,
                     s2b1_conv2_b, s2b1_conv2_shift, nb=8)

    x = _down_block(x, s3b0_conv1_b, s3b0_conv1_shift,
                    s3b0_conv2_b, s3b0_conv2_shift,
                    s3b0_down_b, s3b0_down_shift, nb_s1=16, nb_s2=8)
    x = _plain_block(x, s3b1_conv1_b, s3b1_conv1_shift,
                     s3b1_conv2_b, s3b1_conv2_shift, nb=16)

    return avgpool_head(x, head_b, head_shift)
